# Initial kernel scaffold; baseline (speedup 1.0000x reference)
import jax
import jax.numpy as jnp
from jax.experimental import pallas as pl


def kernel(feat0, pid0, w1_0, b1_0, w2_0, b2_0, feat1, pid1, w1_1, b1_1, w2_1, b2_1, feat2, pid2, w1_2, b1_2, w2_2, b2_2):
    raise NotImplementedError("write your pallas kernel here")


# speedup vs baseline: 1.0457x; 1.0457x over previous
import jax
import jax.numpy as jnp
from jax.experimental import pallas as pl
from jax.experimental.pallas import tpu as pltpu


def _scale_body(f_ref, m_ref, w1_ref, b1_ref, w2_ref, b2_ref, o_ref):
    x = f_ref[0].astype(jnp.bfloat16)
    g = jnp.dot(x, m_ref[...], preferred_element_type=jnp.float32)
    gb = g.astype(jnp.bfloat16)
    h = jax.lax.dot_general(
        gb, w1_ref[...], (((0,), (0,)), ((), ())),
        preferred_element_type=jnp.float32)
    h = jnp.maximum(h + b1_ref[...], 0.0).astype(jnp.bfloat16)
    y = jnp.dot(h, w2_ref[...], preferred_element_type=jnp.float32)
    y = y + b2_ref[...]
    norm = jnp.sqrt(jnp.sum(y * y, axis=-1, keepdims=True))
    o_ref[0] = y / (norm + 1e-7)


def _fused_kernel(f0, m0, w10, b10, w20, b20,
                  f1, m1, w11, b11, w21, b21,
                  f2, m2, w12, b12, w22, b22,
                  o0, o1, o2):
    _scale_body(f0, m0, w10, b10, w20, b20, o0)
    _scale_body(f1, m1, w11, b11, w21, b21, o1)
    _scale_body(f2, m2, w12, b12, w22, b22, o2)


def _onehot_t(pid, hw):
    iota = jax.lax.broadcasted_iota(jnp.int32, (hw, pid.shape[0]), 0)
    return (iota == pid[None, :].astype(jnp.int32)).astype(jnp.bfloat16)


def kernel(feat0, pid0, w1_0, b1_0, w2_0, b2_0,
           feat1, pid1, w1_1, b1_1, w2_1, b2_1,
           feat2, pid2, w1_2, b1_2, w2_2, b2_2):
    B = feat0.shape[0]
    nc = w1_0.shape[1]
    P = pid0.shape[0]

    feats, masks, wts = [], [], []
    flops = 0
    bytes_accessed = 0
    for feat, pid, w1, b1, w2, b2 in (
            (feat0, pid0, w1_0, b1_0, w2_0, b2_0),
            (feat1, pid1, w1_1, b1_1, w2_1, b2_1),
            (feat2, pid2, w1_2, b1_2, w2_2, b2_2)):
        C = feat.shape[1]
        hw = feat.shape[2] * feat.shape[3]
        feats.append(feat.reshape(B, C, hw))
        masks.append(_onehot_t(pid, hw))
        wts.append((w1.astype(jnp.bfloat16), b1.reshape(1, nc),
                    w2.astype(jnp.bfloat16), b2.reshape(1, nc)))
        flops += 2 * B * (C * hw * P + P * C * nc + P * nc * nc) + 5 * B * P * nc
        bytes_accessed += 4 * (B * C * hw + B * P * nc) + 2 * (hw * P + C * nc + nc * nc)

    operands = []
    in_specs = []
    for s in range(3):
        C = feats[s].shape[1]
        hw = feats[s].shape[2]
        w1b, b1r, w2b, b2r = wts[s]
        operands += [feats[s], masks[s], w1b, b1r, w2b, b2r]
        in_specs += [
            pl.BlockSpec((1, C, hw), lambda b: (b, 0, 0)),
            pl.BlockSpec((hw, P), lambda b: (0, 0)),
            pl.BlockSpec((C, nc), lambda b: (0, 0)),
            pl.BlockSpec((1, nc), lambda b: (0, 0)),
            pl.BlockSpec((nc, nc), lambda b: (0, 0)),
            pl.BlockSpec((1, nc), lambda b: (0, 0)),
        ]

    outs = pl.pallas_call(
        _fused_kernel,
        out_shape=[jax.ShapeDtypeStruct((B, P, nc), jnp.float32)] * 3,
        grid=(B,),
        in_specs=in_specs,
        out_specs=[pl.BlockSpec((1, P, nc), lambda b: (b, 0, 0))] * 3,
        compiler_params=pltpu.CompilerParams(
            dimension_semantics=("parallel",)),
        cost_estimate=pl.CostEstimate(
            flops=flops, transcendentals=B * P * 3,
            bytes_accessed=bytes_accessed),
    )(*operands)
    return list(outs)


# speedup vs baseline: 1.2261x; 1.1726x over previous
import jax
import jax.numpy as jnp
from jax.experimental import pallas as pl
from jax.experimental.pallas import tpu as pltpu


def _fused_kernel(p0_ref, p1_ref, p2_ref,
                  f0, w10, b10, w20, b20,
                  f1, w11, b11, w21, b21,
                  f2, w12, b12, w22, b22,
                  o0, o1, o2,
                  m0_s, m1_s, m2_s):
    j = pl.program_id(1)

    @pl.when(j == 0)
    def _build_masks():
        for m_s, p_ref in ((m0_s, p0_ref), (m1_s, p1_ref), (m2_s, p2_ref)):
            hw, npat = m_s.shape
            iota = jax.lax.broadcasted_iota(jnp.int32, (hw, npat), 0)
            m_s[...] = (iota == p_ref[...]).astype(jnp.bfloat16)

    for f, w1, b1, w2, b2, m_s, o in (
            (f0, w10, b10, w20, b20, m0_s, o0),
            (f1, w11, b11, w21, b21, m1_s, o1),
            (f2, w12, b12, w22, b22, m2_s, o2)):
        x = f[0]
        g = jnp.dot(x, m_s[...], preferred_element_type=jnp.float32)
        gb = g.astype(jnp.bfloat16)
        h = jax.lax.dot_general(
            gb, w1[...].astype(jnp.bfloat16), (((0,), (0,)), ((), ())),
            preferred_element_type=jnp.float32)
        h = jnp.maximum(h + b1[...], 0.0).astype(jnp.bfloat16)
        y = jnp.dot(h, w2[...].astype(jnp.bfloat16),
                    preferred_element_type=jnp.float32)
        y = y + b2[...]
        norm = jnp.sqrt(jnp.sum(y * y, axis=-1, keepdims=True))
        o[0] = y / (norm + 1e-7)


def kernel(feat0, pid0, w1_0, b1_0, w2_0, b2_0,
           feat1, pid1, w1_1, b1_1, w2_1, b2_1,
           feat2, pid2, w1_2, b1_2, w2_2, b2_2):
    B = feat0.shape[0]
    nc = w1_0.shape[1]
    P = pid0.shape[0]
    half = B // 2

    pids, feats, wts, hws, cs = [], [], [], [], []
    flops = 0
    bytes_accessed = 0
    for feat, pid, w1, b1, w2, b2 in (
            (feat0, pid0, w1_0, b1_0, w2_0, b2_0),
            (feat1, pid1, w1_1, b1_1, w2_1, b2_1),
            (feat2, pid2, w1_2, b1_2, w2_2, b2_2)):
        C = feat.shape[1]
        hw = feat.shape[2] * feat.shape[3]
        cs.append(C)
        hws.append(hw)
        pids.append(pid.reshape(1, P))
        feats.append(feat.reshape(B, C, hw).astype(jnp.bfloat16))
        wts.append((w1, b1.reshape(1, nc), w2, b2.reshape(1, nc)))
        flops += 2 * B * (C * hw * P + P * C * nc + P * nc * nc) + 5 * B * P * nc
        bytes_accessed += 2 * B * C * hw + 4 * B * P * nc + 4 * (C * nc + nc * nc)

    operands = list(pids)
    in_specs = [pl.BlockSpec((1, P), lambda c, j: (0, 0))] * 3
    for s in range(3):
        C, hw = cs[s], hws[s]
        w1, b1r, w2, b2r = wts[s]
        operands += [feats[s], w1, b1r, w2, b2r]
        in_specs += [
            pl.BlockSpec((1, C, hw), lambda c, j, h=half: (c * h + j, 0, 0)),
            pl.BlockSpec((C, nc), lambda c, j: (0, 0)),
            pl.BlockSpec((1, nc), lambda c, j: (0, 0)),
            pl.BlockSpec((nc, nc), lambda c, j: (0, 0)),
            pl.BlockSpec((1, nc), lambda c, j: (0, 0)),
        ]

    outs = pl.pallas_call(
        _fused_kernel,
        out_shape=[jax.ShapeDtypeStruct((B, P, nc), jnp.float32)] * 3,
        grid=(2, half),
        in_specs=in_specs,
        out_specs=[pl.BlockSpec((1, P, nc),
                                lambda c, j, h=half: (c * h + j, 0, 0))] * 3,
        scratch_shapes=[pltpu.VMEM((hws[s], P), jnp.bfloat16) for s in range(3)],
        compiler_params=pltpu.CompilerParams(
            dimension_semantics=("parallel", "arbitrary")),
        cost_estimate=pl.CostEstimate(
            flops=flops, transcendentals=B * P * 3,
            bytes_accessed=bytes_accessed),
    )(*operands)
    return list(outs)


# speedup vs baseline: 1.9461x; 1.5872x over previous
import jax
import jax.numpy as jnp
from jax.experimental import pallas as pl
from jax.experimental.pallas import tpu as pltpu


def _fused_kernel(p0_ref, p1_ref, p2_ref,
                  f0, w10, b10, w20, b20,
                  f1, w11, b11, w21, b21,
                  f2, w12, b12, w22, b22,
                  o0, o1, o2,
                  m0_s, m1_s, m2_s):
    j = pl.program_id(1)

    @pl.when(j == 0)
    def _build_masks():
        for m_s, p_ref in ((m0_s, p0_ref), (m1_s, p1_ref), (m2_s, p2_ref)):
            npat, hw = m_s.shape
            pid_col = p_ref[...].reshape(npat, 1)
            iota = jax.lax.broadcasted_iota(jnp.int32, (npat, hw), 1)
            m_s[...] = (iota == pid_col).astype(jnp.bfloat16)

    for f, w1, b1, w2, b2, m_s, o in (
            (f0, w10, b10, w20, b20, m0_s, o0),
            (f1, w11, b11, w21, b21, m1_s, o1),
            (f2, w12, b12, w22, b22, m2_s, o2)):
        x = f[0]
        g = jnp.dot(m_s[...], x, preferred_element_type=jnp.float32)
        gb = g.astype(jnp.bfloat16)
        h = jnp.dot(gb, w1[...].astype(jnp.bfloat16),
                    preferred_element_type=jnp.float32)
        h = jnp.maximum(h + b1[...], 0.0).astype(jnp.bfloat16)
        y = jnp.dot(h, w2[...].astype(jnp.bfloat16),
                    preferred_element_type=jnp.float32)
        y = y + b2[...]
        norm = jnp.sqrt(jnp.sum(y * y, axis=-1, keepdims=True))
        o[0] = y / (norm + 1e-7)


def kernel(feat0, pid0, w1_0, b1_0, w2_0, b2_0,
           feat1, pid1, w1_1, b1_1, w2_1, b2_1,
           feat2, pid2, w1_2, b1_2, w2_2, b2_2):
    B = feat0.shape[0]
    nc = w1_0.shape[1]
    P = pid0.shape[0]
    half = B // 2

    pids, feats, wts, hws, cs = [], [], [], [], []
    flops = 0
    bytes_accessed = 0
    for feat, pid, w1, b1, w2, b2 in (
            (feat0, pid0, w1_0, b1_0, w2_0, b2_0),
            (feat1, pid1, w1_1, b1_1, w2_1, b2_1),
            (feat2, pid2, w1_2, b1_2, w2_2, b2_2)):
        C = feat.shape[1]
        hw = feat.shape[2] * feat.shape[3]
        cs.append(C)
        hws.append(hw)
        pids.append(pid.reshape(1, P))
        feats.append(jnp.transpose(feat, (0, 2, 3, 1))
                     .reshape(B, hw, C).astype(jnp.bfloat16))
        wts.append((w1, b1.reshape(1, nc), w2, b2.reshape(1, nc)))
        flops += 2 * B * (P * hw * C + P * C * nc + P * nc * nc) + 5 * B * P * nc
        bytes_accessed += 2 * B * C * hw + 4 * B * P * nc + 4 * (C * nc + nc * nc)

    operands = list(pids)
    in_specs = [pl.BlockSpec((1, P), lambda c, j: (0, 0))] * 3
    for s in range(3):
        C, hw = cs[s], hws[s]
        w1, b1r, w2, b2r = wts[s]
        operands += [feats[s], w1, b1r, w2, b2r]
        in_specs += [
            pl.BlockSpec((1, hw, C), lambda c, j, h=half: (c * h + j, 0, 0)),
            pl.BlockSpec((C, nc), lambda c, j: (0, 0)),
            pl.BlockSpec((1, nc), lambda c, j: (0, 0)),
            pl.BlockSpec((nc, nc), lambda c, j: (0, 0)),
            pl.BlockSpec((1, nc), lambda c, j: (0, 0)),
        ]

    outs = pl.pallas_call(
        _fused_kernel,
        out_shape=[jax.ShapeDtypeStruct((B, P, nc), jnp.float32)] * 3,
        grid=(2, half),
        in_specs=in_specs,
        out_specs=[pl.BlockSpec((1, P, nc),
                                lambda c, j, h=half: (c * h + j, 0, 0))] * 3,
        scratch_shapes=[pltpu.VMEM((P, hws[s]), jnp.bfloat16) for s in range(3)],
        compiler_params=pltpu.CompilerParams(
            dimension_semantics=("parallel", "arbitrary")),
        cost_estimate=pl.CostEstimate(
            flops=flops, transcendentals=B * P * 3,
            bytes_accessed=bytes_accessed),
    )(*operands)
    return list(outs)
